# baseline (speedup 1.0000x reference)
import functools
import jax
import jax.numpy as jnp
from jax.experimental import pallas as pl
from jax.experimental.pallas import tpu as pltpu

BN = 2000
ECHUNK = 1000
KB = 64


def _proj_body(x_ref, w_ref, o_ref):
    o_ref[...] = jnp.dot(x_ref[...], w_ref[...],
                         preferred_element_type=jnp.float32)


def _project(x, w):
    n, d_in = x.shape
    k = w.shape[1]
    return pl.pallas_call(
        _proj_body,
        grid=(n // BN,),
        in_specs=[pl.BlockSpec((BN, d_in), lambda i: (i, 0)),
                  pl.BlockSpec((d_in, k), lambda i: (0, 0))],
        out_specs=pl.BlockSpec((BN, k), lambda i: (i, 0)),
        out_shape=jax.ShapeDtypeStruct((n, k), jnp.float32),
    )(x, w)


def _seg_body(ei_ref, y_ref, out_ref):
    @pl.when(pl.program_id(0) == 0)
    def _():
        out_ref[...] = jnp.zeros(out_ref.shape, jnp.float32)

    def body(j, carry):
        s = ei_ref[0, 0, j]
        d = ei_ref[0, 1, j]
        out_ref[pl.ds(d, 1), :] += y_ref[pl.ds(s, 1), :]
        return carry

    jax.lax.fori_loop(0, ECHUNK, body, 0, unroll=8)


def _segment_sum(ei_blocked, y):
    n, k = y.shape
    nblk = ei_blocked.shape[0]
    return pl.pallas_call(
        _seg_body,
        grid=(nblk,),
        in_specs=[
            pl.BlockSpec((1, 2, ECHUNK), lambda i: (i, 0, 0),
                         memory_space=pltpu.SMEM),
            pl.BlockSpec((n, k), lambda i: (0, 0)),
        ],
        out_specs=pl.BlockSpec((n, k), lambda i: (0, 0)),
        out_shape=jax.ShapeDtypeStruct((n, k), jnp.float32),
        compiler_params=pltpu.CompilerParams(
            vmem_limit_bytes=128 * 1024 * 1024),
    )(ei_blocked, y)


def _cnt_body(ei_ref, cnt_ref):
    @pl.when(pl.program_id(0) == 0)
    def _():
        cnt_ref[...] = jnp.zeros(cnt_ref.shape, jnp.float32)

    def body(j, carry):
        d = ei_ref[0, 1, j]
        cnt_ref[pl.ds(d, 1), :] += 1.0
        return carry

    jax.lax.fori_loop(0, ECHUNK, body, 0, unroll=8)


def _segment_count(ei_blocked, n):
    nblk = ei_blocked.shape[0]
    return pl.pallas_call(
        _cnt_body,
        grid=(nblk,),
        in_specs=[pl.BlockSpec((1, 2, ECHUNK), lambda i: (i, 0, 0),
                              memory_space=pltpu.SMEM)],
        out_specs=pl.BlockSpec((n, 8), lambda i: (0, 0)),
        out_shape=jax.ShapeDtypeStruct((n, 8), jnp.float32),
        compiler_params=pltpu.CompilerParams(
            vmem_limit_bytes=128 * 1024 * 1024),
    )(ei_blocked)


def _combine2_body(sa_ref, ca_ref, sb_ref, cb_ref, x_ref, w_ref, b_ref,
                   o_ref, *, relu):
    agg_a = sa_ref[...] / jnp.maximum(ca_ref[:, :1], 1.0)
    agg_b = sb_ref[...] / jnp.maximum(cb_ref[:, :1], 1.0)
    out = agg_a + agg_b + jnp.dot(x_ref[...], w_ref[...],
                                  preferred_element_type=jnp.float32)
    out = out + b_ref[...]
    if relu:
        out = jnp.maximum(out, 0.0)
    o_ref[...] = out


def _combine1_body(sa_ref, ca_ref, x_ref, w_ref, b_ref, o_ref, *, relu):
    agg_a = sa_ref[...] / jnp.maximum(ca_ref[:, :1], 1.0)
    out = agg_a + jnp.dot(x_ref[...], w_ref[...],
                          preferred_element_type=jnp.float32)
    out = out + b_ref[...]
    if relu:
        out = jnp.maximum(out, 0.0)
    o_ref[...] = out


def _combine2(sa, ca, sb, cb, x, w, b, relu):
    n, k = sa.shape
    d_in = x.shape[1]
    return pl.pallas_call(
        functools.partial(_combine2_body, relu=relu),
        grid=(n // BN,),
        in_specs=[pl.BlockSpec((BN, k), lambda i: (i, 0)),
                  pl.BlockSpec((BN, 8), lambda i: (i, 0)),
                  pl.BlockSpec((BN, k), lambda i: (i, 0)),
                  pl.BlockSpec((BN, 8), lambda i: (i, 0)),
                  pl.BlockSpec((BN, d_in), lambda i: (i, 0)),
                  pl.BlockSpec((d_in, k), lambda i: (0, 0)),
                  pl.BlockSpec((1, k), lambda i: (0, 0))],
        out_specs=pl.BlockSpec((BN, k), lambda i: (i, 0)),
        out_shape=jax.ShapeDtypeStruct((n, k), jnp.float32),
    )(sa, ca, sb, cb, x, w, b.reshape(1, k))


def _combine1(sa, ca, x, w, b, relu):
    n, k = sa.shape
    d_in = x.shape[1]
    return pl.pallas_call(
        functools.partial(_combine1_body, relu=relu),
        grid=(n // BN,),
        in_specs=[pl.BlockSpec((BN, k), lambda i: (i, 0)),
                  pl.BlockSpec((BN, 8), lambda i: (i, 0)),
                  pl.BlockSpec((BN, d_in), lambda i: (i, 0)),
                  pl.BlockSpec((d_in, k), lambda i: (0, 0)),
                  pl.BlockSpec((1, k), lambda i: (0, 0))],
        out_specs=pl.BlockSpec((BN, k), lambda i: (i, 0)),
        out_shape=jax.ShapeDtypeStruct((n, k), jnp.float32),
    )(sa, ca, x, w, b.reshape(1, k))


def _block_edges(ei):
    e = ei.shape[1]
    return ei.reshape(2, e // ECHUNK, ECHUNK).transpose(1, 0, 2)


def kernel(x_movie, x_director, x_actor, ei_md, ei_dm, ei_ma, ei_am,
           W1_md_l, W1_md_r, b1_md, W1_dm_l, W1_dm_r, b1_dm,
           W1_ma_l, W1_ma_r, b1_ma, W1_am_l, W1_am_r, b1_am,
           W2_md_l, W2_md_r, b2_md, W2_dm_l, W2_dm_r, b2_dm,
           W2_ma_l, W2_ma_r, b2_ma, W2_am_l, W2_am_r, b2_am):
    e_md = _block_edges(ei_md)
    e_dm = _block_edges(ei_dm)
    e_ma = _block_edges(ei_ma)
    e_am = _block_edges(ei_am)

    n = x_movie.shape[0]
    c_dm = _segment_count(e_dm, n)
    c_am = _segment_count(e_am, n)
    c_md = _segment_count(e_md, n)
    c_ma = _segment_count(e_ma, n)

    s_dm = _segment_sum(e_dm, _project(x_director, W1_dm_l))
    s_am = _segment_sum(e_am, _project(x_actor, W1_am_l))
    s_md = _segment_sum(e_md, _project(x_movie, W1_md_l))
    s_ma = _segment_sum(e_ma, _project(x_movie, W1_ma_l))

    h_movie = _combine2(s_dm, c_dm, s_am, c_am, x_movie,
                        W1_dm_r + W1_am_r, b1_dm + b1_am, relu=True)
    h_director = _combine1(s_md, c_md, x_director, W1_md_r, b1_md, relu=True)
    h_actor = _combine1(s_ma, c_ma, x_actor, W1_ma_r, b1_ma, relu=True)

    s2_dm = _segment_sum(e_dm, _project(h_director, W2_dm_l))
    s2_am = _segment_sum(e_am, _project(h_actor, W2_am_l))
    s2_md = _segment_sum(e_md, _project(h_movie, W2_md_l))
    s2_ma = _segment_sum(e_ma, _project(h_movie, W2_ma_l))

    o_movie = _combine2(s2_dm, c_dm, s2_am, c_am, h_movie,
                        W2_dm_r + W2_am_r, b2_dm + b2_am, relu=False)
    o_director = _combine1(s2_md, c_md, h_director, W2_md_r, b2_md,
                           relu=False)
    o_actor = _combine1(s2_ma, c_ma, h_actor, W2_ma_r, b2_ma, relu=False)
    return (o_movie, o_director, o_actor)
